# Initial kernel scaffold; baseline (speedup 1.0000x reference)
import jax
import jax.numpy as jnp
from jax.experimental import pallas as pl


def kernel(nodes, senders, receivers, W1, b1, W2, b2):
    raise NotImplementedError("write your pallas kernel here")


# speedup vs baseline: 7.9924x; 7.9924x over previous
import jax
import jax.numpy as jnp
from jax import lax
from jax.experimental import pallas as pl
from jax.experimental.pallas import tpu as pltpu
from jax.experimental.pallas import tpu_sc as plsc

N_NODES = 10000
N_EDGES = 320000
D = 128
DH = D // 2

NC = 2
NS = 16
EPT = N_EDGES // NS
C = 80
NB = 2
CPB = EPT // C // NB
NPAD = 10240
RPT = NPAD // NS
RCH = 128


def _sc_body(ncols_hbm, send_hbm, recv_hbm, out_hbm,
             sidx, ridx, rows, obuf, acc, sems):
    cid = lax.axis_index("c")
    sid = lax.axis_index("s")

    def _zb(i, carry):
        obuf[i // (DH // 16), pl.ds((i % (DH // 16)) * 16, 16)] = (
            jnp.zeros((16,), jnp.float32))
        return carry
    lax.fori_loop(0, RCH * DH // 16, _zb, 0)
    r0 = sid * RPT
    for k in range(RPT // RCH):
        pltpu.sync_copy(obuf, acc.at[pl.ds(r0 + k * RCH, RCH)])
    plsc.subcore_barrier()

    for b in range(NB):
        pltpu.sync_copy(send_hbm.at[sid, b], sidx)
        pltpu.sync_copy(recv_hbm.at[sid, b], ridx)
        pltpu.async_copy(ncols_hbm.at[cid].at[sidx.at[0]], rows.at[0], sems.at[0])

        def _edge(j, carry):
            jm = lax.rem(j, 2)
            jn = lax.rem(j + 1, 2)

            @pl.when(j < CPB - 1)
            def _fire():
                pltpu.async_copy(ncols_hbm.at[cid].at[sidx.at[j + 1]],
                                 rows.at[jn], sems.at[jn])

            pltpu.make_async_copy(ncols_hbm.at[cid].at[sidx.at[j]], rows.at[jm],
                                  sems.at[jm]).wait()
            pltpu.sync_copy(rows.at[jm], acc.at[ridx.at[j]], add=True)
            return carry
        lax.fori_loop(0, CPB, _edge, 0)
    plsc.subcore_barrier()

    for k in range(RPT // RCH):
        pltpu.sync_copy(acc.at[pl.ds(r0 + k * RCH, RCH)], obuf)
        pltpu.sync_copy(obuf, out_hbm.at[cid, pl.ds(r0 + k * RCH, RCH)])


_sc_aggregate = pl.kernel(
    _sc_body,
    out_type=jax.ShapeDtypeStruct((NC, NPAD, DH), jnp.float32),
    mesh=plsc.VectorSubcoreMesh(core_axis_name="c", subcore_axis_name="s",
                                num_cores=NC, num_subcores=NS),
    compiler_params=pltpu.CompilerParams(use_tc_tiling_on_sc=False),
    scratch_types=[
        pltpu.VMEM((CPB, C), jnp.int32),
        pltpu.VMEM((CPB, C), jnp.int32),
        pltpu.VMEM((2, C, DH), jnp.float32),
        pltpu.VMEM((RCH, DH), jnp.float32),
        pltpu.VMEM_SHARED((NPAD, DH), jnp.float32),
        pltpu.SemaphoreType.DMA((2,)),
    ],
)


def _mlp_body(part_ref, nodes_ref, w1_ref, b1_ref, w2_ref, b2_ref, out_ref):
    h = jnp.concatenate([part_ref[0], part_ref[1]], axis=1) + nodes_ref[...]
    h1 = jnp.maximum(
        jnp.dot(h, w1_ref[...], preferred_element_type=jnp.float32)
        + b1_ref[...], 0.0)
    out_ref[...] = (jnp.dot(h1, w2_ref[...], preferred_element_type=jnp.float32)
                    + b2_ref[...])


_BLK = 400


def _tc_mlp(partials, nodes, W1, b1, W2, b2):
    grid = N_NODES // _BLK
    return pl.pallas_call(
        _mlp_body,
        grid=(grid,),
        in_specs=[
            pl.BlockSpec((NC, _BLK, DH), lambda i: (0, i, 0)),
            pl.BlockSpec((_BLK, D), lambda i: (i, 0)),
            pl.BlockSpec((D, D), lambda i: (0, 0)),
            pl.BlockSpec((1, D), lambda i: (0, 0)),
            pl.BlockSpec((D, D), lambda i: (0, 0)),
            pl.BlockSpec((1, D), lambda i: (0, 0)),
        ],
        out_specs=pl.BlockSpec((_BLK, D), lambda i: (i, 0)),
        out_shape=jax.ShapeDtypeStruct((N_NODES, D), jnp.float32),
    )(partials, nodes, W1, b1, W2, b2)


def kernel(nodes, senders, receivers, W1, b1, W2, b2):
    ncols = jnp.stack([nodes[:, :DH], nodes[:, DH:]])
    send4d = senders.reshape(NS, NB, CPB, C)
    recv4d = receivers.reshape(NS, NB, CPB, C)
    partials = _sc_aggregate(ncols, send4d, recv4d)
    return _tc_mlp(partials, nodes, W1, b1.reshape(1, D), W2, b2.reshape(1, D))


# speedup vs baseline: 9.9318x; 1.2427x over previous
import jax
import jax.numpy as jnp
from jax import lax
from jax.experimental import pallas as pl
from jax.experimental.pallas import tpu as pltpu
from jax.experimental.pallas import tpu_sc as plsc

N_NODES = 10000
N_EDGES = 320000
D = 128
DH = D // 2

NC = 2
NS = 16
EPT = N_EDGES // NS
C = 80
NB = 5
CPB = EPT // C // NB
NR = 4
NPAD = 10240
RPT = NPAD // NS
RCH = 64


def _sc_body(ncols_hbm, send_hbm, recv_hbm, out_hbm,
             sidx, ridx, rows, obuf, acc, sems, isems):
    cid = lax.axis_index("c")
    sid = lax.axis_index("s")
    tbl = ncols_hbm.at[cid]

    def _zb(i, carry):
        obuf[i // (DH // 16), pl.ds((i % (DH // 16)) * 16, 16)] = (
            jnp.zeros((16,), jnp.float32))
        return carry
    lax.fori_loop(0, RCH * DH // 16, _zb, 0)
    r0 = sid * RPT
    for k in range(RPT // RCH):
        pltpu.sync_copy(obuf, acc.at[pl.ds(r0 + k * RCH, RCH)])
    plsc.subcore_barrier()

    pltpu.sync_copy(send_hbm.at[sid, 0], sidx.at[0])
    pltpu.sync_copy(recv_hbm.at[sid, 0], ridx.at[0])
    for t in range(NR - 1):
        pltpu.async_copy(tbl.at[sidx.at[0, t]], rows.at[t], sems.at[t])

    for b in range(NB):
        sl, nsl = b % 2, (b + 1) % 2
        off = (b * CPB) % NR

        if b + 1 < NB:
            pltpu.async_copy(send_hbm.at[sid, b + 1], sidx.at[nsl],
                             isems.at[0])
            pltpu.async_copy(recv_hbm.at[sid, b + 1], ridx.at[nsl],
                             isems.at[1])

        def _main(j, carry, sl=sl, off=off):
            jm = lax.rem(j + off, NR)
            jf = lax.rem(j + NR - 1 + off, NR)
            pltpu.make_async_copy(tbl.at[sidx.at[sl, j]], rows.at[jm],
                                  sems.at[jm]).wait()
            pltpu.sync_copy(rows.at[jm], acc.at[ridx.at[sl, j]], add=True)
            pltpu.async_copy(tbl.at[sidx.at[sl, j + NR - 1]], rows.at[jf],
                             sems.at[jf])
            return carry
        lax.fori_loop(0, CPB - (NR - 1), _main, 0)

        if b + 1 < NB:
            pltpu.make_async_copy(send_hbm.at[sid, b + 1], sidx.at[nsl],
                                  isems.at[0]).wait()
            pltpu.make_async_copy(recv_hbm.at[sid, b + 1], ridx.at[nsl],
                                  isems.at[1]).wait()

        def _tail(j, carry, sl=sl, nsl=nsl, off=off, last=(b + 1 == NB)):
            jm = lax.rem(j + off, NR)
            jf = lax.rem(j + NR - 1 + off, NR)
            pltpu.make_async_copy(tbl.at[sidx.at[sl, j]], rows.at[jm],
                                  sems.at[jm]).wait()
            pltpu.sync_copy(rows.at[jm], acc.at[ridx.at[sl, j]], add=True)
            if not last:
                pltpu.async_copy(tbl.at[sidx.at[nsl, j + NR - 1 - CPB]],
                                 rows.at[jf], sems.at[jf])
            return carry
        lax.fori_loop(CPB - (NR - 1), CPB, _tail, 0)
    plsc.subcore_barrier()

    for k in range(RPT // RCH):
        pltpu.sync_copy(acc.at[pl.ds(r0 + k * RCH, RCH)], obuf)
        pltpu.sync_copy(obuf, out_hbm.at[cid, pl.ds(r0 + k * RCH, RCH)])


_sc_aggregate = pl.kernel(
    _sc_body,
    out_type=jax.ShapeDtypeStruct((NC, NPAD, DH), jnp.float32),
    mesh=plsc.VectorSubcoreMesh(core_axis_name="c", subcore_axis_name="s",
                                num_cores=NC, num_subcores=NS),
    compiler_params=pltpu.CompilerParams(use_tc_tiling_on_sc=False),
    scratch_types=[
        pltpu.VMEM((2, CPB, C), jnp.int32),
        pltpu.VMEM((2, CPB, C), jnp.int32),
        pltpu.VMEM((NR, C, DH), jnp.float32),
        pltpu.VMEM((RCH, DH), jnp.float32),
        pltpu.VMEM_SHARED((NPAD, DH), jnp.float32),
        pltpu.SemaphoreType.DMA((NR,)),
        pltpu.SemaphoreType.DMA((2,)),
    ],
)


def _mlp_body(part_ref, nodes_ref, w1_ref, b1_ref, w2_ref, b2_ref, out_ref):
    h = jnp.concatenate([part_ref[0], part_ref[1]], axis=1) + nodes_ref[...]
    h1 = jnp.maximum(
        jnp.dot(h, w1_ref[...], preferred_element_type=jnp.float32)
        + b1_ref[...], 0.0)
    out_ref[...] = (jnp.dot(h1, w2_ref[...], preferred_element_type=jnp.float32)
                    + b2_ref[...])


_BLK = 400


def _tc_mlp(partials, nodes, W1, b1, W2, b2):
    grid = N_NODES // _BLK
    return pl.pallas_call(
        _mlp_body,
        grid=(grid,),
        in_specs=[
            pl.BlockSpec((NC, _BLK, DH), lambda i: (0, i, 0)),
            pl.BlockSpec((_BLK, D), lambda i: (i, 0)),
            pl.BlockSpec((D, D), lambda i: (0, 0)),
            pl.BlockSpec((1, D), lambda i: (0, 0)),
            pl.BlockSpec((D, D), lambda i: (0, 0)),
            pl.BlockSpec((1, D), lambda i: (0, 0)),
        ],
        out_specs=pl.BlockSpec((_BLK, D), lambda i: (i, 0)),
        out_shape=jax.ShapeDtypeStruct((N_NODES, D), jnp.float32),
    )(partials, nodes, W1, b1, W2, b2)


def kernel(nodes, senders, receivers, W1, b1, W2, b2):
    ncols = jnp.stack([nodes[:, :DH], nodes[:, DH:]])
    send4d = senders.reshape(NS, NB, CPB, C)
    recv4d = receivers.reshape(NS, NB, CPB, C)
    partials = _sc_aggregate(ncols, send4d, recv4d)
    return _tc_mlp(partials, nodes, W1, b1.reshape(1, D), W2, b2.reshape(1, D))
